# baseline (speedup 1.0000x reference)
import functools

import jax
import jax.numpy as jnp
from jax import lax
from jax.experimental import pallas as pl
from jax.experimental.pallas import tpu as pltpu
from jax.experimental.pallas import tpu_sc as plsc

NUM_ENTITY = 100000
NUM_REL = 500
EMBED_DIM = 128
BATCH = 16384

_INFO = plsc.get_sparse_core_info()
_NC, _NS, _L = _INFO.num_cores, _INFO.num_subcores, _INFO.num_lanes
_NW = _NC * _NS
_BPW = BATCH // _NW


_CHUNK = _BPW // 2
_NBUF = 3


def _sc_kernel(sub_hbm, rel_hbm, obj_hbm, node_hbm, rel_t_hbm, out_hbm,
               sub_v, rel_v, obj_v, rows_v, rel_sh,
               gsem0, gsem1, gsem2, ssem0, ssem1, ssem2):
    sid = lax.axis_index("s")
    wid = sid * _NC + lax.axis_index("c")
    base = wid * _BPW

    idx_bufs = (sub_v, rel_v, obj_v)
    gsems = (gsem0, gsem1, gsem2)
    ssems = (ssem0, ssem1, ssem2)

    pltpu.sync_copy(sub_hbm.at[pl.ds(base, _BPW)], sub_v)

    items = [(0, 0), (0, 1), (2, 0), (2, 1), (1, 0), (1, 1)]

    def gather(i, b):
        t, h = items[i]
        idx = idx_bufs[t].at[pl.ds(h * _CHUNK, _CHUNK)]
        if t == 1:
            src = rel_sh.at[idx]
        else:
            src = node_hbm.at[pl.ds(1, NUM_ENTITY)].at[idx]
        pltpu.async_copy(src, rows_v.at[b], gsems[b])

    n = len(items)

    def wait_gather(b):
        pltpu.make_async_copy(node_hbm.at[sub_v.at[pl.ds(0, _CHUNK)]],
                              rows_v.at[b], gsems[b]).wait()

    def wait_scatter(b):
        pltpu.make_async_copy(rows_v.at[b], out_hbm.at[0, pl.ds(0, _CHUNK)],
                              ssems[b]).wait()

    gather(0, 0)

    pltpu.sync_copy(rel_hbm.at[pl.ds(base, _BPW)], rel_v)
    pltpu.sync_copy(obj_hbm.at[pl.ds(base, _BPW)], obj_v)

    @pl.when(sid == 0)
    def _stage_rel():
        pltpu.sync_copy(rel_t_hbm.at[pl.ds(1, NUM_REL)], rel_sh)

    gather(1, 1)

    for i in range(n):
        b = i % _NBUF
        t, h = items[i]
        wait_gather(b)
        pltpu.async_copy(rows_v.at[b],
                         out_hbm.at[t, pl.ds(base + h * _CHUNK, _CHUNK)],
                         ssems[b])
        j = i + 2
        if j < n:
            if items[j][0] == 1 and items[j - 1][0] != 1:
                plsc.subcore_barrier()
            bj = j % _NBUF
            if j >= _NBUF:
                wait_scatter(bj)
            gather(j, bj)

    for i in range(max(0, n - _NBUF), n):
        wait_scatter(i % _NBUF)


@jax.jit
def _run(sub_idx, rel_idx, obj_idx, node_table, rel_table):
    k = functools.partial(
        pl.kernel,
        mesh=plsc.VectorSubcoreMesh(core_axis_name="c", subcore_axis_name="s"),
        out_type=jax.ShapeDtypeStruct((3, BATCH, EMBED_DIM), jnp.float32),
        compiler_params=pltpu.CompilerParams(
            use_tc_tiling_on_sc=False,
            disable_bounds_checks=True,
            disable_semaphore_checks=True,
            skip_device_barrier=True,
        ),
        scratch_types=[
            pltpu.VMEM((_BPW,), jnp.int32),
            pltpu.VMEM((_BPW,), jnp.int32),
            pltpu.VMEM((_BPW,), jnp.int32),
            pltpu.VMEM((_NBUF, _CHUNK, EMBED_DIM), jnp.float32),
            pltpu.VMEM_SHARED((NUM_REL, EMBED_DIM), jnp.float32),
            pltpu.SemaphoreType.DMA,
            pltpu.SemaphoreType.DMA,
            pltpu.SemaphoreType.DMA,
            pltpu.SemaphoreType.DMA,
            pltpu.SemaphoreType.DMA,
            pltpu.SemaphoreType.DMA,
        ],
    )(_sc_kernel)
    return k(sub_idx, rel_idx, obj_idx, node_table, rel_table)


def kernel(sub_idx, rel_idx, obj_idx, node_table, rel_table):
    return _run(sub_idx.astype(jnp.int32), rel_idx.astype(jnp.int32),
                obj_idx.astype(jnp.int32), node_table, rel_table)


# speedup vs baseline: 1.0135x; 1.0135x over previous
import functools

import jax
import jax.numpy as jnp
from jax import lax
from jax.experimental import pallas as pl
from jax.experimental.pallas import tpu as pltpu
from jax.experimental.pallas import tpu_sc as plsc

NUM_ENTITY = 100000
NUM_REL = 500
EMBED_DIM = 128
BATCH = 16384

_INFO = plsc.get_sparse_core_info()
_NC, _NS, _L = _INFO.num_cores, _INFO.num_subcores, _INFO.num_lanes
_NW = _NC * _NS
_BPW = BATCH // _NW

_NQ = 4
_CHUNK = _BPW // _NQ
_NBUF = 7
_AHEAD = 3

_ITEMS = ([(0, q) for q in range(_NQ)] + [(2, q) for q in range(_NQ)]
          + [(1, q) for q in range(_NQ)])
_FIRST_REL = _NQ * 2


def _sc_kernel(sub_hbm, rel_hbm, obj_hbm, node_hbm, rel_t_hbm, out_hbm,
               sub_v, rel_v, obj_v, rows_v, rel_sh, *sems):
    sid = lax.axis_index("s")
    wid = sid * _NC + lax.axis_index("c")
    base = wid * _BPW

    idx_bufs = (sub_v, rel_v, obj_v)
    gsems = sems[:_NBUF]
    ssems = sems[_NBUF:]
    n = len(_ITEMS)

    def gather(i, b):
        t, q = _ITEMS[i]
        idx = idx_bufs[t].at[pl.ds(q * _CHUNK, _CHUNK)]
        if t == 1:
            src = rel_sh.at[idx]
        else:
            src = node_hbm.at[pl.ds(1, NUM_ENTITY)].at[idx]
        pltpu.async_copy(src, rows_v.at[b], gsems[b])

    def wait_gather(b):
        pltpu.make_async_copy(node_hbm.at[sub_v.at[pl.ds(0, _CHUNK)]],
                              rows_v.at[b], gsems[b]).wait()

    def wait_scatter(b):
        pltpu.make_async_copy(rows_v.at[b], out_hbm.at[0, pl.ds(0, _CHUNK)],
                              ssems[b]).wait()

    pltpu.sync_copy(sub_hbm.at[pl.ds(base, _BPW)], sub_v)
    for j in range(2):
        gather(j, j)

    pltpu.sync_copy(obj_hbm.at[pl.ds(base, _BPW)], obj_v)
    pltpu.sync_copy(rel_hbm.at[pl.ds(base, _BPW)], rel_v)

    @pl.when(sid == 0)
    def _stage_rel():
        pltpu.sync_copy(rel_t_hbm.at[pl.ds(1, NUM_REL)], rel_sh)

    for j in range(2, _AHEAD):
        gather(j, j)

    for i in range(n):
        b = i % _NBUF
        t, q = _ITEMS[i]
        wait_gather(b)
        pltpu.async_copy(rows_v.at[b],
                         out_hbm.at[t, pl.ds(base + q * _CHUNK, _CHUNK)],
                         ssems[b])
        j = i + _AHEAD
        if j < n:
            if j == _FIRST_REL:
                plsc.subcore_barrier()
            bj = j % _NBUF
            if j >= _NBUF:
                wait_scatter(bj)
            gather(j, bj)

    for i in range(max(0, n - _NBUF), n):
        wait_scatter(i % _NBUF)


@jax.jit
def _run(sub_idx, rel_idx, obj_idx, node_table, rel_table):
    k = functools.partial(
        pl.kernel,
        mesh=plsc.VectorSubcoreMesh(core_axis_name="c", subcore_axis_name="s"),
        out_type=jax.ShapeDtypeStruct((3, BATCH, EMBED_DIM), jnp.float32),
        compiler_params=pltpu.CompilerParams(use_tc_tiling_on_sc=False),
        scratch_types=[
            pltpu.VMEM((_BPW,), jnp.int32),
            pltpu.VMEM((_BPW,), jnp.int32),
            pltpu.VMEM((_BPW,), jnp.int32),
            pltpu.VMEM((_NBUF, _CHUNK, EMBED_DIM), jnp.float32),
            pltpu.VMEM_SHARED((NUM_REL, EMBED_DIM), jnp.float32),
        ] + [pltpu.SemaphoreType.DMA] * (2 * _NBUF),
    )(_sc_kernel)
    return k(sub_idx, rel_idx, obj_idx, node_table, rel_table)


def kernel(sub_idx, rel_idx, obj_idx, node_table, rel_table):
    return _run(sub_idx.astype(jnp.int32), rel_idx.astype(jnp.int32),
                obj_idx.astype(jnp.int32), node_table, rel_table)
